# baseline (speedup 1.0000x reference)
import functools

import jax
import jax.numpy as jnp
from jax.experimental import pallas as pl
from jax.experimental.pallas import tpu as pltpu

EPS = 1e-5


def _make_conv_stats_kernel(img_w, wo, L):
    def _body(x_ref, w_ref, y_ref, sum_ref, ssq_ref):
        x = x_ref[0].astype(jnp.bfloat16)
        cout = y_ref.shape[1]
        acc = jnp.zeros((cout, L), jnp.float32)
        for kh in range(3):
            for kw in range(3):
                off = kh * img_w + kw
                acc = acc + jnp.dot(
                    w_ref[kh * 3 + kw], x[:, off:off + L],
                    preferred_element_type=jnp.float32)

        y_ref[0] = acc.astype(jnp.bfloat16)

        col = jax.lax.broadcasted_iota(jnp.int32, (1, L), 1)
        mask = (col % img_w) < wo
        accm = jnp.where(mask, acc, 0.0)
        sum_ref[0] = jnp.sum(accm, axis=1, keepdims=True)
        ssq_ref[0] = jnp.sum(accm * acc, axis=1, keepdims=True)

    return _body


def _make_bn_relu_kernel(img_w, ho, wo, m_valid):
    def _body(y_ref, sum_ref, ssq_ref, g_ref, b_ref, o_ref):
        tot = jnp.sum(sum_ref[...], axis=0)
        tsq = jnp.sum(ssq_ref[...], axis=0)
        mean = tot / m_valid
        var = jnp.maximum(tsq / m_valid - mean * mean, 0.0)
        inv = jax.lax.rsqrt(var + EPS)
        scale = g_ref[...] * inv
        shift = b_ref[...] - mean * scale

        z = y_ref[0].astype(jnp.float32) * scale + shift
        z = jnp.maximum(z, 0.0)
        for j in range(ho):
            o_ref[0, :, j * wo:(j + 1) * wo] = z[:, j * img_w:j * img_w + wo]

    return _body


@functools.partial(jax.jit, static_argnames=())
def kernel(x_nchw, w_oihw, bias, gamma, beta):
    del bias
    n, cin, h, w = x_nchw.shape
    cout = w_oihw.shape[0]
    ho, wo = h - 2, w - 2
    L = ho * w - (w - wo)

    x_flat = x_nchw.reshape(n, cin, h * w)

    w_taps = jnp.transpose(w_oihw, (2, 3, 0, 1)).reshape(9, cout, cin)
    w_taps = w_taps.astype(jnp.bfloat16)
    g_col = gamma.reshape(cout, 1)
    b_col = beta.reshape(cout, 1)

    y, sums, ssqs = pl.pallas_call(
        _make_conv_stats_kernel(w, wo, L),
        out_shape=(
            jax.ShapeDtypeStruct((n, cout, L), jnp.bfloat16),
            jax.ShapeDtypeStruct((n, cout, 1), jnp.float32),
            jax.ShapeDtypeStruct((n, cout, 1), jnp.float32),
        ),
        grid=(n,),
        in_specs=[
            pl.BlockSpec((1, cin, h * w), lambda i: (i, 0, 0)),
            pl.BlockSpec((9, cout, cin), lambda i: (0, 0, 0)),
        ],
        out_specs=(
            pl.BlockSpec((1, cout, L), lambda i: (i, 0, 0)),
            pl.BlockSpec((1, cout, 1), lambda i: (i, 0, 0)),
            pl.BlockSpec((1, cout, 1), lambda i: (i, 0, 0)),
        ),
        compiler_params=pltpu.CompilerParams(dimension_semantics=("parallel",)),
    )(x_flat, w_taps)

    out = pl.pallas_call(
        _make_bn_relu_kernel(w, ho, wo, float(n * ho * wo)),
        out_shape=jax.ShapeDtypeStruct((n, cout, ho * wo), jnp.float32),
        grid=(n,),
        in_specs=[
            pl.BlockSpec((1, cout, L), lambda i: (i, 0, 0)),
            pl.BlockSpec((n, cout, 1), lambda i: (0, 0, 0)),
            pl.BlockSpec((n, cout, 1), lambda i: (0, 0, 0)),
            pl.BlockSpec((cout, 1), lambda i: (0, 0)),
            pl.BlockSpec((cout, 1), lambda i: (0, 0)),
        ],
        out_specs=pl.BlockSpec((1, cout, ho * wo), lambda i: (i, 0, 0)),
        compiler_params=pltpu.CompilerParams(dimension_semantics=("parallel",)),
    )(y, sums, ssqs, g_col, b_col)

    return out.reshape(n, cout, ho, wo)
